# Initial kernel scaffold; baseline (speedup 1.0000x reference)
import jax
import jax.numpy as jnp
from jax.experimental import pallas as pl


def kernel(inputs, pos_table):
    raise NotImplementedError("write your pallas kernel here")


# speedup vs baseline: 1.6708x; 1.6708x over previous
import jax
import jax.numpy as jnp
from jax.experimental import pallas as pl


def _add_kernel(x_ref, p_ref, o_ref):
    o_ref[...] = x_ref[...] + p_ref[...][None, :, :]


def kernel(inputs, pos_table):
    batch, seq_len, d_model = inputs.shape
    blk_s = 256
    grid = (seq_len // blk_s, batch)
    return pl.pallas_call(
        _add_kernel,
        grid=grid,
        in_specs=[
            pl.BlockSpec((1, blk_s, d_model), lambda i, j: (j, i, 0)),
            pl.BlockSpec((blk_s, d_model), lambda i, j: (i, 0)),
        ],
        out_specs=pl.BlockSpec((1, blk_s, d_model), lambda i, j: (j, i, 0)),
        out_shape=jax.ShapeDtypeStruct(inputs.shape, inputs.dtype),
    )(inputs, pos_table)


# speedup vs baseline: 1.8456x; 1.1046x over previous
import jax
import jax.numpy as jnp
from jax.experimental import pallas as pl


def _add_kernel(x_ref, p_ref, o_ref):
    o_ref[...] = x_ref[...] + p_ref[...][None, :, :]


def kernel(inputs, pos_table):
    batch, seq_len, d_model = inputs.shape
    blk_s = 512
    grid = (seq_len // blk_s, batch)
    return pl.pallas_call(
        _add_kernel,
        grid=grid,
        in_specs=[
            pl.BlockSpec((1, blk_s, d_model), lambda i, j: (j, i, 0)),
            pl.BlockSpec((blk_s, d_model), lambda i, j: (i, 0)),
        ],
        out_specs=pl.BlockSpec((1, blk_s, d_model), lambda i, j: (j, i, 0)),
        out_shape=jax.ShapeDtypeStruct(inputs.shape, inputs.dtype),
    )(inputs, pos_table)


# speedup vs baseline: 1.9693x; 1.0670x over previous
import jax
import jax.numpy as jnp
from jax.experimental import pallas as pl


def _add_kernel(x_ref, p_ref, o_ref):
    o_ref[...] = x_ref[...] + p_ref[...][None, :, :]


def kernel(inputs, pos_table):
    batch, seq_len, d_model = inputs.shape
    blk_s = 1024
    grid = (seq_len // blk_s, batch)
    return pl.pallas_call(
        _add_kernel,
        grid=grid,
        in_specs=[
            pl.BlockSpec((1, blk_s, d_model), lambda i, j: (j, i, 0)),
            pl.BlockSpec((blk_s, d_model), lambda i, j: (i, 0)),
        ],
        out_specs=pl.BlockSpec((1, blk_s, d_model), lambda i, j: (j, i, 0)),
        out_shape=jax.ShapeDtypeStruct(inputs.shape, inputs.dtype),
    )(inputs, pos_table)
